# baseline (speedup 1.0000x reference)
import jax
import jax.numpy as jnp
from jax import lax
from jax.experimental import pallas as pl
from jax.experimental.pallas import tpu as pltpu
from jax.experimental.pallas import tpu_sc as plsc

BATCH = 16384
FIELDS = 26
VOCAB = 100000
DIM = 16
DENSE = 13
OUT_W = FIELDS * DIM + DENSE

NC = 2
NS = 16
NW = NC * NS
ROWS_PER_W = BATCH // NW
CHUNK = 128
NCHUNK = ROWS_PER_W // CHUNK


def _body(xs_t_hbm, dense_hbm, tables_hbm, out_hbm, xs_all, rows_v, dense_v,
          sem, wsem):
    wid = lax.axis_index("s") * NC + lax.axis_index("c")
    base = wid * ROWS_PER_W

    for f in range(FIELDS):
        pltpu.make_async_copy(
            xs_t_hbm.at[pl.ds(f * BATCH + base, ROWS_PER_W)],
            xs_all.at[pl.ds(f * ROWS_PER_W, ROWS_PER_W)],
            sem,
        ).start()
    for f in range(FIELDS):
        pltpu.make_async_copy(
            xs_t_hbm.at[pl.ds(f * BATCH + base, ROWS_PER_W)],
            xs_all.at[pl.ds(f * ROWS_PER_W, ROWS_PER_W)],
            sem,
        ).wait()

    def chunk_step(c, _):
        row0 = base + c * CHUNK
        dense_in = pltpu.make_async_copy(
            dense_hbm.at[pl.ds(row0, CHUNK), :], dense_v, sem)
        dense_in.start()
        for f in range(FIELDS):
            pltpu.make_async_copy(
                tables_hbm.at[f].at[xs_all.at[pl.ds(f * ROWS_PER_W + c * CHUNK,
                                                    CHUNK)]],
                rows_v.at[f],
                sem,
            ).start()
        dense_in.wait()
        dense_out = pltpu.make_async_copy(
            dense_v, out_hbm.at[pl.ds(row0, CHUNK), pl.ds(FIELDS * DIM, DENSE)],
            wsem)
        dense_out.start()
        for f in range(FIELDS):
            pltpu.make_async_copy(
                tables_hbm.at[f].at[xs_all.at[pl.ds(f * ROWS_PER_W + c * CHUNK,
                                                    CHUNK)]],
                rows_v.at[f],
                sem,
            ).wait()
            pltpu.make_async_copy(
                rows_v.at[f],
                out_hbm.at[pl.ds(row0, CHUNK), pl.ds(f * DIM, DIM)],
                wsem,
            ).start()
        dense_out.wait()
        for f in range(FIELDS):
            pltpu.make_async_copy(
                rows_v.at[f],
                out_hbm.at[pl.ds(row0, CHUNK), pl.ds(f * DIM, DIM)],
                wsem,
            ).wait()
        return ()

    lax.fori_loop(0, NCHUNK, chunk_step, ())


@jax.jit
def _run(xs_t_flat, x_dense, tables):
    mesh = plsc.VectorSubcoreMesh(
        core_axis_name="c", subcore_axis_name="s", num_cores=NC, num_subcores=NS
    )
    return pl.kernel(
        _body,
        out_type=jax.ShapeDtypeStruct((BATCH, OUT_W), jnp.float32),
        mesh=mesh,
        compiler_params=pltpu.CompilerParams(use_tc_tiling_on_sc=False),
        scratch_types=[
            pltpu.VMEM((FIELDS * ROWS_PER_W,), jnp.int32),
            pltpu.VMEM((FIELDS, CHUNK, DIM), jnp.float32),
            pltpu.VMEM((CHUNK, DENSE), jnp.float32),
            pltpu.SemaphoreType.DMA,
            pltpu.SemaphoreType.DMA,
        ],
    )(xs_t_flat, x_dense, tables)


def kernel(x_sparse, x_dense, tables):
    xs_t_flat = x_sparse.T.reshape(-1)
    return _run(xs_t_flat, x_dense, tables)
